# baseline (speedup 1.0000x reference)
import functools

import jax
import jax.numpy as jnp
from jax import lax
from jax.experimental import pallas as pl
from jax.experimental.pallas import tpu as pltpu
from jax.experimental.pallas import tpu_sc as plsc

N_NODES = 10000
N_EDGES = 320000
D_FEAT = 128
D_PACK = D_FEAT // 2

NC = 2
NS = 16
L = 16
NW = NC * NS
EDGES_PER_TILE = N_EDGES // NW
BLK = 40
NBLK = EDGES_PER_TILE // BLK
GBUF = 5
SBUF = 2
SUP = 10
NSUP = NBLK // SUP
ROW_CHUNK = 624
ROW_TAIL = N_NODES - NS * ROW_CHUNK

_mesh = plsc.VectorSubcoreMesh(core_axis_name="c", subcore_axis_name="s")


@functools.partial(
    pl.kernel,
    out_type=jax.ShapeDtypeStruct((NC, N_NODES, D_FEAT), jnp.float32),
    mesh=_mesh,
    scratch_types=[
        pltpu.VMEM((EDGES_PER_TILE,), jnp.int32),
        pltpu.VMEM((EDGES_PER_TILE,), jnp.float32),
        [pltpu.VMEM((BLK,), jnp.int32) for _ in range(GBUF)],
        [pltpu.VMEM((BLK, D_PACK), jnp.int32) for _ in range(GBUF)],
        [pltpu.VMEM((BLK, D_FEAT), jnp.float32) for _ in range(SBUF)],
        pltpu.VMEM_SHARED((N_NODES, D_FEAT), jnp.float32),
        [pltpu.SemaphoreType.DMA for _ in range(GBUF)],
        [pltpu.SemaphoreType.DMA for _ in range(GBUF)],
        [pltpu.SemaphoreType.DMA for _ in range(SBUF)],
    ],
    compiler_params=pltpu.CompilerParams(needs_layout_passes=False,
                                         use_tc_tiling_on_sc=False),
)
def _sc_aggregate(node_perm_hbm, zeros_hbm, packed_hbm, src_hbm, dst_hbm,
                  attr_hbm, part_hbm, srcv, attrv, dstv, packed, scaled,
                  accum, gsem, dsem, ssem):
    c = lax.axis_index("c")
    s = lax.axis_index("s")
    wid = c * NS + s

    rsl = pl.ds(s * ROW_CHUNK, ROW_CHUNK)
    tsl = pl.ds(NS * ROW_CHUNK, ROW_TAIL)

    @pl.when(c == 0)
    def _():
        pltpu.sync_copy(node_perm_hbm.at[rsl], accum.at[rsl])

        @pl.when(s == 0)
        def _():
            pltpu.sync_copy(node_perm_hbm.at[tsl], accum.at[tsl])

    @pl.when(c != 0)
    def _():
        pltpu.sync_copy(zeros_hbm.at[rsl], accum.at[rsl])

        @pl.when(s == 0)
        def _():
            pltpu.sync_copy(zeros_hbm.at[tsl], accum.at[tsl])

    ebase = wid * EDGES_PER_TILE
    pltpu.sync_copy(src_hbm.at[pl.ds(ebase, EDGES_PER_TILE)], srcv)
    pltpu.sync_copy(attr_hbm.at[pl.ds(ebase, EDGES_PER_TILE)], attrv)

    plsc.subcore_barrier()

    def start_dst(j, g):
        pltpu.async_copy(dst_hbm.at[pl.ds(ebase + j * BLK, BLK)], dstv[g],
                         dsem[g])

    def start_gather(j, g):
        pltpu.async_copy(packed_hbm.at[srcv.at[pl.ds(j * BLK, BLK)]],
                         packed[g], gsem[g])

    def finish_block(j, g, k, drain, pref_dst, pref_gather):
        off = j * BLK
        pltpu.make_async_copy(
            packed_hbm.at[srcv.at[pl.ds(off, BLK)]], packed[g], gsem[g]).wait()
        if drain:
            pltpu.make_async_copy(scaled[k], accum.at[dstv[g]], ssem[k]).wait()
            if pref_dst:
                start_dst(j - SBUF + GBUF, (g - SBUF) % GBUF)

        @pl.loop(0, BLK, unroll=4)
        def _scale(e):
            bc = plsc.load_gather(attrv, [jnp.full((L,), off, jnp.int32) + e])
            bcp = plsc.pack(bc, bc, format=plsc.PackFormat.INTERLEAVED,
                            preferred_element_type=jnp.bfloat16)
            for cc in range(D_PACK // L):
                pk = packed[g][e, pl.ds(cc * L, L)]
                vb = plsc.bitcast(pk, jnp.bfloat16)
                prod = vb * bcp
                x, y = plsc.unpack(prod, format=plsc.PackFormat.INTERLEAVED,
                                   preferred_element_type=jnp.float32)
                scaled[k][e, pl.ds(cc * 2 * L, L)] = x
                scaled[k][e, pl.ds(cc * 2 * L + L, L)] = y

        pltpu.make_async_copy(dst_hbm.at[pl.ds(ebase + off, BLK)], dstv[g],
                              dsem[g]).wait()
        pltpu.async_copy(scaled[k], accum.at[dstv[g]], ssem[k], add=True)
        if pref_gather:
            start_gather(j + GBUF, g)

    for g in range(GBUF):
        start_dst(g, g)
        start_gather(g, g)

    for k in range(SUP):
        finish_block(k, k % GBUF, k % SBUF, k >= SBUF, k >= SBUF, True)

    @pl.loop(1, NSUP - 1)
    def _super(t):
        j0 = t * SUP
        for k in range(SUP):
            finish_block(j0 + k, k % GBUF, k % SBUF, True, True, True)

    j0 = (NSUP - 1) * SUP
    for k in range(SUP):
        finish_block(j0 + k, k % GBUF, k % SBUF, True,
                     j0 + k - SBUF + GBUF < NBLK, j0 + k + GBUF < NBLK)
    for k in range(SBUF):
        pltpu.make_async_copy(scaled[k], accum.at[dstv[k]], ssem[k]).wait()

    plsc.subcore_barrier()
    pltpu.sync_copy(accum.at[rsl], part_hbm.at[c, rsl])

    @pl.when(s == 0)
    def _():
        pltpu.sync_copy(accum.at[tsl], part_hbm.at[c, tsl])


def _combine_body(p_ref, o_ref):
    s = p_ref[0] + p_ref[1]
    b = s.shape[0]
    o_ref[...] = s.reshape(b, 4, 2, 16).transpose(0, 1, 3, 2).reshape(b, 128)


_combine = pl.pallas_call(
    _combine_body,
    out_shape=jax.ShapeDtypeStruct((N_NODES, D_FEAT), jnp.float32),
    grid=(10,),
    in_specs=[pl.BlockSpec((NC, N_NODES // 10, D_FEAT), lambda i: (0, i, 0))],
    out_specs=pl.BlockSpec((N_NODES // 10, D_FEAT), lambda i: (i, 0)),
)


@jax.jit
def kernel(node_feat, edge_index, edge_attr):
    src = edge_index[0].astype(jnp.int32)
    dst = edge_index[1].astype(jnp.int32)
    zeros = jnp.zeros_like(node_feat)
    u = lax.bitcast_convert_type(node_feat.astype(jnp.bfloat16), jnp.uint16)
    lo = u[:, 0::2].astype(jnp.uint32)
    hi = u[:, 1::2].astype(jnp.uint32)
    packed = lax.bitcast_convert_type(lo | (hi << 16), jnp.int32)
    node_perm = (node_feat.reshape(N_NODES, 4, 16, 2)
                 .transpose(0, 1, 3, 2).reshape(N_NODES, D_FEAT))
    part = _sc_aggregate(node_perm, zeros, packed, src, dst, edge_attr)
    return _combine(part)


# speedup vs baseline: 1.2082x; 1.2082x over previous
import functools

import jax
import jax.numpy as jnp
from jax import lax
from jax.experimental import pallas as pl
from jax.experimental.pallas import tpu as pltpu
from jax.experimental.pallas import tpu_sc as plsc

N_NODES = 10000
N_EDGES = 320000
D_FEAT = 128
D_HALF = D_FEAT // 2
D_PACK = D_HALF // 2

NC = 2
NS = 16
L = 16
EDGES_PER_TILE = N_EDGES // NS
BLK = 80
NBLK = EDGES_PER_TILE // BLK
GBUF = 5
SBUF = 2
SUP = 10
NSUP = NBLK // SUP
ROW_CHUNK = 624
ROW_TAIL = N_NODES - NS * ROW_CHUNK

_mesh = plsc.VectorSubcoreMesh(core_axis_name="c", subcore_axis_name="s")


@functools.partial(
    pl.kernel,
    out_type=jax.ShapeDtypeStruct((NC, N_NODES, D_HALF), jnp.float32),
    mesh=_mesh,
    scratch_types=[
        pltpu.VMEM((EDGES_PER_TILE,), jnp.int32),
        pltpu.VMEM((EDGES_PER_TILE,), jnp.float32),
        [pltpu.VMEM((BLK,), jnp.int32) for _ in range(GBUF)],
        [pltpu.VMEM((BLK, D_PACK), jnp.int32) for _ in range(GBUF)],
        [pltpu.VMEM((BLK, D_HALF), jnp.float32) for _ in range(SBUF)],
        pltpu.VMEM_SHARED((N_NODES, D_PACK), jnp.int32),
        pltpu.VMEM_SHARED((N_NODES, D_HALF), jnp.float32),
        [pltpu.SemaphoreType.DMA for _ in range(GBUF)],
        [pltpu.SemaphoreType.DMA for _ in range(GBUF)],
        [pltpu.SemaphoreType.DMA for _ in range(SBUF)],
    ],
    compiler_params=pltpu.CompilerParams(needs_layout_passes=False,
                                         use_tc_tiling_on_sc=False),
)
def _sc_aggregate(node_perm_hbm, packed_hbm, src_hbm, dst_hbm, attr_hbm,
                  part_hbm, srcv, attrv, dstv, packed, scaled, table, accum,
                  gsem, dsem, ssem):
    c = lax.axis_index("c")
    s = lax.axis_index("s")

    rsl = pl.ds(s * ROW_CHUNK, ROW_CHUNK)
    tsl = pl.ds(NS * ROW_CHUNK, ROW_TAIL)
    pltpu.sync_copy(node_perm_hbm.at[c, rsl], accum.at[rsl])
    pltpu.sync_copy(packed_hbm.at[c, rsl], table.at[rsl])

    @pl.when(s == 0)
    def _():
        pltpu.sync_copy(node_perm_hbm.at[c, tsl], accum.at[tsl])
        pltpu.sync_copy(packed_hbm.at[c, tsl], table.at[tsl])

    ebase = s * EDGES_PER_TILE
    pltpu.sync_copy(src_hbm.at[pl.ds(ebase, EDGES_PER_TILE)], srcv)
    pltpu.sync_copy(attr_hbm.at[pl.ds(ebase, EDGES_PER_TILE)], attrv)

    plsc.subcore_barrier()

    def start_dst(j, g):
        pltpu.async_copy(dst_hbm.at[pl.ds(ebase + j * BLK, BLK)], dstv[g],
                         dsem[g])

    def start_gather(j, g):
        pltpu.async_copy(table.at[srcv.at[pl.ds(j * BLK, BLK)]],
                         packed[g], gsem[g])

    def finish_block(j, g, k, drain, pref_dst, pref_gather):
        off = j * BLK
        pltpu.make_async_copy(
            table.at[srcv.at[pl.ds(off, BLK)]], packed[g], gsem[g]).wait()
        if drain:
            pltpu.make_async_copy(scaled[k], accum.at[dstv[g]], ssem[k]).wait()
            if pref_dst:
                start_dst(j - SBUF + GBUF, (g - SBUF) % GBUF)

        @pl.loop(0, BLK, unroll=8)
        def _scale(e):
            bc = plsc.load_gather(attrv, [jnp.full((L,), off, jnp.int32) + e])
            for cc in range(D_PACK // L):
                pk = packed[g][e, pl.ds(cc * L, L)]
                x = plsc.bitcast(lax.shift_left(pk, 16), jnp.float32)
                y = plsc.bitcast(
                    lax.bitwise_and(pk, jnp.int32(-65536)), jnp.float32)
                scaled[k][e, pl.ds(cc * 2 * L, L)] = x * bc
                scaled[k][e, pl.ds(cc * 2 * L + L, L)] = y * bc

        pltpu.make_async_copy(dst_hbm.at[pl.ds(ebase + off, BLK)], dstv[g],
                              dsem[g]).wait()
        pltpu.async_copy(scaled[k], accum.at[dstv[g]], ssem[k], add=True)
        if pref_gather:
            start_gather(j + GBUF, g)

    for g in range(GBUF):
        start_dst(g, g)
        start_gather(g, g)

    for k in range(SUP):
        finish_block(k, k % GBUF, k % SBUF, k >= SBUF, k >= SBUF, True)

    @pl.loop(1, NSUP - 1)
    def _super(t):
        j0 = t * SUP
        for k in range(SUP):
            finish_block(j0 + k, k % GBUF, k % SBUF, True, True, True)

    j0 = (NSUP - 1) * SUP
    for k in range(SUP):
        finish_block(j0 + k, k % GBUF, k % SBUF, True,
                     j0 + k - SBUF + GBUF < NBLK, j0 + k + GBUF < NBLK)
    for k in range(SBUF):
        pltpu.make_async_copy(scaled[k], accum.at[dstv[k]], ssem[k]).wait()

    plsc.subcore_barrier()
    pltpu.sync_copy(accum.at[rsl], part_hbm.at[c, rsl])

    @pl.when(s == 0)
    def _():
        pltpu.sync_copy(accum.at[tsl], part_hbm.at[c, tsl])


def _unperm(h):
    b = h.shape[0]
    return h.reshape(b, 2, 2, 16).transpose(0, 1, 3, 2).reshape(b, D_HALF)


def _combine_body(p_ref, o_ref):
    o_ref[...] = jnp.concatenate([_unperm(p_ref[0]), _unperm(p_ref[1])],
                                 axis=1)


_combine = pl.pallas_call(
    _combine_body,
    out_shape=jax.ShapeDtypeStruct((N_NODES, D_FEAT), jnp.float32),
    grid=(10,),
    in_specs=[pl.BlockSpec((NC, N_NODES // 10, D_HALF), lambda i: (0, i, 0))],
    out_specs=pl.BlockSpec((N_NODES // 10, D_FEAT), lambda i: (i, 0)),
)


def _pack_half(half):
    u = lax.bitcast_convert_type(half.astype(jnp.bfloat16), jnp.uint16)
    lo = u[:, 0::2].astype(jnp.uint32)
    hi = u[:, 1::2].astype(jnp.uint32)
    return lax.bitcast_convert_type(lo | (hi << 16), jnp.int32)


def _perm_half(half):
    return (half.reshape(N_NODES, 2, 16, 2).transpose(0, 1, 3, 2)
            .reshape(N_NODES, D_HALF))


@jax.jit
def kernel(node_feat, edge_index, edge_attr):
    src = edge_index[0].astype(jnp.int32)
    dst = edge_index[1].astype(jnp.int32)
    halves = [node_feat[:, :D_HALF], node_feat[:, D_HALF:]]
    packed = jnp.stack([_pack_half(h) for h in halves])
    node_perm = jnp.stack([_perm_half(h) for h in halves])
    part = _sc_aggregate(node_perm, packed, src, dst, edge_attr)
    return _combine(part)


# speedup vs baseline: 3.8233x; 3.1644x over previous
import functools

import jax
import jax.numpy as jnp
from jax import lax
from jax.experimental import pallas as pl
from jax.experimental.pallas import tpu as pltpu
from jax.experimental.pallas import tpu_sc as plsc

N_NODES = 10000
N_EDGES = 320000
D_FEAT = 128

NC = 2
NS = 16
L = 16
NW = NC * NS
EDGES_PER_TILE = N_EDGES // NW
BLK = 40
NBUF = 5
NBLK = EDGES_PER_TILE // BLK
NSUP = NBLK // NBUF
ROW_CHUNK = 624
ROW_TAIL = N_NODES - NS * ROW_CHUNK

_mesh = plsc.VectorSubcoreMesh(core_axis_name="c", subcore_axis_name="s")


@functools.partial(
    pl.kernel,
    out_type=jax.ShapeDtypeStruct((NC, N_NODES, D_FEAT), jnp.float32),
    mesh=_mesh,
    scratch_types=[
        pltpu.VMEM((EDGES_PER_TILE,), jnp.int32),
        pltpu.VMEM((EDGES_PER_TILE,), jnp.float32),
        [pltpu.VMEM((BLK,), jnp.int32) for _ in range(NBUF)],
        [pltpu.VMEM((BLK, D_FEAT), jnp.float32) for _ in range(NBUF)],
        pltpu.VMEM_SHARED((N_NODES, D_FEAT), jnp.float32),
        [pltpu.SemaphoreType.DMA for _ in range(NBUF)],
        [pltpu.SemaphoreType.DMA for _ in range(NBUF)],
        [pltpu.SemaphoreType.DMA for _ in range(NBUF)],
    ],
    compiler_params=pltpu.CompilerParams(needs_layout_passes=False),
)
def _sc_aggregate(node_hbm, zeros_hbm, src_hbm, dst_hbm, attr_hbm, part_hbm,
                  srcv, attrv, dstv, rows, accum, gsem, dsem, ssem):
    c = lax.axis_index("c")
    s = lax.axis_index("s")
    wid = c * NS + s

    rsl = pl.ds(s * ROW_CHUNK, ROW_CHUNK)
    tsl = pl.ds(NS * ROW_CHUNK, ROW_TAIL)

    @pl.when(c == 0)
    def _():
        pltpu.sync_copy(node_hbm.at[rsl], accum.at[rsl])

        @pl.when(s == 0)
        def _():
            pltpu.sync_copy(node_hbm.at[tsl], accum.at[tsl])

    @pl.when(c != 0)
    def _():
        pltpu.sync_copy(zeros_hbm.at[rsl], accum.at[rsl])

        @pl.when(s == 0)
        def _():
            pltpu.sync_copy(zeros_hbm.at[tsl], accum.at[tsl])

    ebase = wid * EDGES_PER_TILE
    pltpu.sync_copy(src_hbm.at[pl.ds(ebase, EDGES_PER_TILE)], srcv)
    pltpu.sync_copy(attr_hbm.at[pl.ds(ebase, EDGES_PER_TILE)], attrv)

    plsc.subcore_barrier()

    def start_block(j, k):
        off = j * BLK
        pltpu.async_copy(dst_hbm.at[pl.ds(ebase + off, BLK)], dstv[k], dsem[k])
        pltpu.async_copy(node_hbm.at[srcv.at[pl.ds(off, BLK)]], rows[k],
                         gsem[k])

    def finish_block(j, k):
        off = j * BLK
        pltpu.make_async_copy(node_hbm.at[srcv.at[pl.ds(off, BLK)]], rows[k],
                              gsem[k]).wait()

        @plsc.parallel_loop(0, BLK, unroll=8)
        def _scale(e):
            bc = plsc.load_gather(
                attrv, [jnp.full((L,), off, jnp.int32) + e])
            for cc in range(D_FEAT // L):
                sl = pl.ds(cc * L, L)
                rows[k][e, sl] = rows[k][e, sl] * bc

        pltpu.make_async_copy(dst_hbm.at[pl.ds(ebase + off, BLK)], dstv[k],
                              dsem[k]).wait()
        pltpu.async_copy(rows[k], accum.at[dstv[k]], ssem[k], add=True)

    def drain_scatter(k):
        pltpu.make_async_copy(rows[k], accum.at[dstv[k]], ssem[k]).wait()

    for k in range(NBUF):
        start_block(k, k)

    @pl.loop(0, NSUP - 1)
    def _super(t):
        j0 = t * NBUF
        for k in range(NBUF):
            finish_block(j0 + k, k)
        for k in range(NBUF):
            drain_scatter(k)
            start_block(j0 + NBUF + k, k)

    for k in range(NBUF):
        finish_block((NSUP - 1) * NBUF + k, k)
    for k in range(NBUF):
        drain_scatter(k)

    plsc.subcore_barrier()
    pltpu.sync_copy(accum.at[rsl], part_hbm.at[c, rsl])

    @pl.when(s == 0)
    def _():
        pltpu.sync_copy(accum.at[tsl], part_hbm.at[c, tsl])


def _combine_body(p_ref, o_ref):
    o_ref[...] = p_ref[0] + p_ref[1]


_combine = pl.pallas_call(
    _combine_body,
    out_shape=jax.ShapeDtypeStruct((N_NODES, D_FEAT), jnp.float32),
    grid=(10,),
    in_specs=[pl.BlockSpec((NC, N_NODES // 10, D_FEAT), lambda i: (0, i, 0))],
    out_specs=pl.BlockSpec((N_NODES // 10, D_FEAT), lambda i: (i, 0)),
)


@jax.jit
def kernel(node_feat, edge_index, edge_attr):
    src = edge_index[0].astype(jnp.int32)
    dst = edge_index[1].astype(jnp.int32)
    zeros = jnp.zeros_like(node_feat)
    part = _sc_aggregate(node_feat, zeros, src, dst, edge_attr)
    return _combine(part)
